# baseline (speedup 1.0000x reference)
import functools

import jax
import jax.numpy as jnp
from jax import lax
from jax.experimental import pallas as pl
from jax.experimental.pallas import tpu as pltpu
from jax.experimental.pallas import tpu_sc as plsc

_HIGH = jax.lax.Precision.HIGHEST


def _u_body(x_ref, we_ref, wef_ref, wg_ref, wc_ref, u_ref):
    w_s = jnp.dot(we_ref[...], wef_ref[...], precision=_HIGH)
    w_y = jnp.dot(wg_ref[...], wc_ref[...], precision=_HIGH)
    w2 = jnp.concatenate([w_s, w_y], axis=1)
    u_ref[...] = jax.lax.dot_general(
        w2, x_ref[...], (((0,), (1,)), ((), ())), precision=_HIGH)


def _fin_body(agg_ref, nd_ref, be_ref, wef_ref, bef_ref, bg_ref, wc_ref,
              bc_ref, out_ref):
    n = out_ref.shape[1]
    bias_s = jnp.sum(be_ref[0, :] * wef_ref[:, 0]) + bef_ref[0, 0]
    bias_y = jnp.sum(bg_ref[0, :] * wc_ref[:, 0]) + bc_ref[0, 0]
    nd = nd_ref[...]
    o_s = ((agg_ref[0, 0] + agg_ref[1, 0]) * nd + bias_s)[:n]
    o_y = ((agg_ref[0, 1] + agg_ref[1, 1]) * nd + bias_y)[:n]
    out_ref[...] = jnp.stack([o_s, o_y])[:, :, None]


def _rsqrt16(x):
    i = lax.bitcast_convert_type(x, jnp.int32)
    y = lax.bitcast_convert_type(jnp.int32(0x5F3759DF) - (i >> 1), jnp.float32)
    for _ in range(3):
        y = y * (1.5 - 0.5 * x * y * y)
    return jnp.where(x > 0.5, y, 0.0)


def _make_sc_kernel(NP, EP):
    R = NP // 16
    G2 = R // 16
    mesh = plsc.VectorSubcoreMesh(core_axis_name="c", subcore_axis_name="s")

    @functools.partial(
        pl.kernel,
        out_type=[
            jax.ShapeDtypeStruct((2, 2, NP), jnp.float32),
            jax.ShapeDtypeStruct((NP,), jnp.float32),
        ],
        mesh=mesh,
        scratch_types=[
            pltpu.VMEM((EP // 16,), jnp.int32),
            pltpu.VMEM((EP // 16,), jnp.int32),
            pltpu.VMEM((EP // 32,), jnp.int32),
            pltpu.VMEM((EP // 32,), jnp.int32),
            pltpu.VMEM((EP // 16,), jnp.float32),
            pltpu.VMEM((EP // 32,), jnp.float32),
            pltpu.VMEM((EP // 32,), jnp.float32),
            pltpu.VMEM((R,), jnp.float32),
            pltpu.VMEM((R,), jnp.float32),
            pltpu.VMEM((R,), jnp.float32),
            pltpu.VMEM((R,), jnp.float32),
            pltpu.VMEM((R,), jnp.float32),
            pltpu.VMEM((R,), jnp.float32),
            pltpu.VMEM((R,), jnp.float32),
            pltpu.VMEM_SHARED((NP,), jnp.float32),
            pltpu.VMEM_SHARED((NP,), jnp.float32),
            pltpu.VMEM_SHARED((NP,), jnp.float32),
            pltpu.VMEM_SHARED((NP,), jnp.float32),
            pltpu.VMEM_SHARED((NP,), jnp.float32),
            pltpu.VMEM_SHARED((NP,), jnp.float32),
        ],
    )
    def sc_kernel(u_hbm, src_hbm, dst_hbm, ones_hbm, zeros_hbm,
                  agg_out, nd_out,
                  sidx, didx, sidx3, didx3, onesv, msg0, msg1,
                  dov, div, u0v, u1v, us0v, us1v, ndv,
                  dego_s, degi_s, us0_s, us1_s, agg0_s, agg1_s):
        c = lax.axis_index("c")
        s = lax.axis_index("s")
        rows = pl.ds(s * R, R)

        pltpu.sync_copy(ones_hbm, onesv)
        pltpu.sync_copy(zeros_hbm.at[rows], dego_s.at[rows])
        pltpu.sync_copy(zeros_hbm.at[rows], degi_s.at[rows])
        pltpu.sync_copy(zeros_hbm.at[rows], agg0_s.at[rows])
        pltpu.sync_copy(zeros_hbm.at[rows], agg1_s.at[rows])
        plsc.subcore_barrier()

        B1 = EP // 16
        p1sl = pl.ds(s * B1, B1)
        pltpu.sync_copy(src_hbm.at[p1sl], sidx)
        pltpu.sync_copy(dst_hbm.at[p1sl], didx)
        pltpu.sync_copy(onesv, dego_s.at[sidx], add=True)
        pltpu.sync_copy(onesv, degi_s.at[didx], add=True)
        plsc.subcore_barrier()

        pltpu.sync_copy(dego_s.at[rows], dov)
        pltpu.sync_copy(degi_s.at[rows], div)
        pltpu.sync_copy(u_hbm.at[0, rows], u0v)
        pltpu.sync_copy(u_hbm.at[1, rows], u1v)

        def norm_g(g, carry):
            sl = pl.ds(g * 16, 16)
            nsrc = _rsqrt16(dov[sl])
            ndv[sl] = _rsqrt16(div[sl])
            us0v[sl] = u0v[sl] * nsrc
            us1v[sl] = u1v[sl] * nsrc
            return carry
        lax.fori_loop(0, G2, norm_g, 0)
        pltpu.sync_copy(us0v, us0_s.at[rows])
        pltpu.sync_copy(us1v, us1_s.at[rows])

        @pl.when(c == 0)
        def _():
            pltpu.sync_copy(ndv, nd_out.at[rows])
        plsc.subcore_barrier()

        wid = s * 2 + c
        B3 = EP // 32
        p3sl = pl.ds(wid * B3, B3)
        pltpu.sync_copy(src_hbm.at[p3sl], sidx3)
        pltpu.sync_copy(dst_hbm.at[p3sl], didx3)
        pltpu.sync_copy(us0_s.at[sidx3], msg0)
        pltpu.sync_copy(us1_s.at[sidx3], msg1)
        pltpu.sync_copy(msg0, agg0_s.at[didx3], add=True)
        pltpu.sync_copy(msg1, agg1_s.at[didx3], add=True)
        plsc.subcore_barrier()

        pltpu.sync_copy(agg0_s.at[rows], agg_out.at[c, 0, rows])
        pltpu.sync_copy(agg1_s.at[rows], agg_out.at[c, 1, rows])

    return sc_kernel


def kernel(x, edge_index, W_est, b_est, W_est_fc, b_est_fc, W_gnn, b_gnn,
           W_cls, b_cls):
    N, D = x.shape
    E = edge_index.shape[1]

    NP = 256 * ((N + 255) // 256)
    EP = 256 * ((E + 255) // 256)
    if EP > E and NP == N:
        NP += 256

    npad = EP - E
    if npad:
        pad = N + (jnp.arange(npad, dtype=jnp.int32) % (NP - N))
        src = jnp.concatenate([edge_index[0], pad])
        dst = jnp.concatenate([edge_index[1], pad])
    else:
        src = edge_index[0]
        dst = edge_index[1]

    BR = 2048
    u = pl.pallas_call(
        _u_body,
        grid=(NP // BR,),
        in_specs=[
            pl.BlockSpec((BR, D), lambda i: (i, 0)),
            pl.BlockSpec((D, D), lambda i: (0, 0)),
            pl.BlockSpec((D, 1), lambda i: (0, 0)),
            pl.BlockSpec((D, D), lambda i: (0, 0)),
            pl.BlockSpec((D, 1), lambda i: (0, 0)),
        ],
        out_specs=pl.BlockSpec((2, BR), lambda i: (0, i)),
        out_shape=jax.ShapeDtypeStruct((2, NP), jnp.float32),
    )(x, W_est, W_est_fc, W_gnn, W_cls)

    ones = jnp.ones((EP // 16,), jnp.float32)
    zeros = jnp.zeros((NP,), jnp.float32)
    agg, nd = _make_sc_kernel(NP, EP)(u, src, dst, ones, zeros)

    out2 = pl.pallas_call(
        _fin_body,
        out_shape=jax.ShapeDtypeStruct((2, N, 1), jnp.float32),
    )(agg, nd, b_est.reshape(1, -1), W_est_fc, b_est_fc.reshape(1, 1),
      b_gnn.reshape(1, -1), W_cls, b_cls.reshape(1, 1))

    return (out2[1], out2[0])


# speedup vs baseline: 1.3370x; 1.3370x over previous
import functools

import jax
import jax.numpy as jnp
from jax import lax
from jax.experimental import pallas as pl
from jax.experimental.pallas import tpu as pltpu
from jax.experimental.pallas import tpu_sc as plsc

_HIGH = jax.lax.Precision.HIGHEST


def _u_body(x_ref, we_ref, wef_ref, wg_ref, wc_ref, u_ref):
    w_s = jax.lax.dot_general(
        wef_ref[...], we_ref[...], (((1,), (1,)), ((), ())), precision=_HIGH)
    w_y = jax.lax.dot_general(
        wc_ref[...], wg_ref[...], (((1,), (1,)), ((), ())), precision=_HIGH)
    w2t = jnp.concatenate([w_s, w_y], axis=0)
    u_ref[...] = jax.lax.dot_general(
        w2t, x_ref[...], (((1,), (1,)), ((), ())), precision=_HIGH)


def _fin_body(agg_ref, nd_ref, be_ref, wef_ref, bef_ref, bg_ref, wc_ref,
              bc_ref, out_ref):
    bias_s = jnp.sum(be_ref[0, :] * wef_ref[0, :]) + bef_ref[0, 0]
    bias_y = jnp.sum(bg_ref[0, :] * wc_ref[0, :]) + bc_ref[0, 0]
    nd = nd_ref[...]
    o_s = (agg_ref[0, 0] + agg_ref[1, 0]) * nd + bias_s
    o_y = (agg_ref[0, 1] + agg_ref[1, 1]) * nd + bias_y
    out_ref[...] = jnp.stack([o_s, o_y])


def _rsqrt16(x):
    i = lax.bitcast_convert_type(x, jnp.int32)
    y = lax.bitcast_convert_type(jnp.int32(0x5F3759DF) - (i >> 1), jnp.float32)
    for _ in range(3):
        y = y * (1.5 - 0.5 * x * y * y)
    return jnp.where(x > 0.5, y, 0.0)


def _make_sc_kernel(NP, EP):
    R = NP // 16
    G2 = R // 16
    mesh = plsc.VectorSubcoreMesh(core_axis_name="c", subcore_axis_name="s")

    @functools.partial(
        pl.kernel,
        out_type=[
            jax.ShapeDtypeStruct((2, 2, NP), jnp.float32),
            jax.ShapeDtypeStruct((NP,), jnp.float32),
        ],
        mesh=mesh,
        scratch_types=[
            pltpu.VMEM((EP // 16,), jnp.int32),
            pltpu.VMEM((EP // 16,), jnp.int32),
            pltpu.VMEM((EP // 32,), jnp.int32),
            pltpu.VMEM((EP // 32,), jnp.int32),
            pltpu.VMEM((EP // 16,), jnp.float32),
            pltpu.VMEM((EP // 32,), jnp.float32),
            pltpu.VMEM((EP // 32,), jnp.float32),
            pltpu.VMEM((R,), jnp.float32),
            pltpu.VMEM((R,), jnp.float32),
            pltpu.VMEM((R,), jnp.float32),
            pltpu.VMEM((R,), jnp.float32),
            pltpu.VMEM((R,), jnp.float32),
            pltpu.VMEM((R,), jnp.float32),
            pltpu.VMEM((R,), jnp.float32),
            pltpu.VMEM_SHARED((NP,), jnp.float32),
            pltpu.VMEM_SHARED((NP,), jnp.float32),
            pltpu.VMEM_SHARED((NP,), jnp.float32),
            pltpu.VMEM_SHARED((NP,), jnp.float32),
            pltpu.VMEM_SHARED((NP,), jnp.float32),
            pltpu.VMEM_SHARED((NP,), jnp.float32),
        ],
    )
    def sc_kernel(u_hbm, ei_hbm, ones_hbm, zeros_hbm,
                  agg_out, nd_out,
                  sidx, didx, sidx3, didx3, onesv, msg0, msg1,
                  dov, div, u0v, u1v, us0v, us1v, ndv,
                  dego_s, degi_s, us0_s, us1_s, agg0_s, agg1_s):
        c = lax.axis_index("c")
        s = lax.axis_index("s")
        rows = pl.ds(s * R, R)

        pltpu.sync_copy(ones_hbm, onesv)
        pltpu.sync_copy(zeros_hbm.at[rows], dego_s.at[rows])
        pltpu.sync_copy(zeros_hbm.at[rows], degi_s.at[rows])
        pltpu.sync_copy(zeros_hbm.at[rows], agg0_s.at[rows])
        pltpu.sync_copy(zeros_hbm.at[rows], agg1_s.at[rows])
        plsc.subcore_barrier()

        B1 = EP // 16
        pltpu.sync_copy(ei_hbm.at[pl.ds(s * B1, B1)], sidx)
        pltpu.sync_copy(ei_hbm.at[pl.ds(EP + s * B1, B1)], didx)
        pltpu.sync_copy(onesv, dego_s.at[sidx], add=True)
        pltpu.sync_copy(onesv, degi_s.at[didx], add=True)
        plsc.subcore_barrier()

        pltpu.sync_copy(dego_s.at[rows], dov)
        pltpu.sync_copy(degi_s.at[rows], div)
        pltpu.sync_copy(u_hbm.at[0, rows], u0v)
        pltpu.sync_copy(u_hbm.at[1, rows], u1v)

        def norm_g(g, carry):
            sl = pl.ds(g * 16, 16)
            nsrc = _rsqrt16(dov[sl])
            ndv[sl] = _rsqrt16(div[sl])
            us0v[sl] = u0v[sl] * nsrc
            us1v[sl] = u1v[sl] * nsrc
            return carry
        lax.fori_loop(0, G2, norm_g, 0)
        pltpu.sync_copy(us0v, us0_s.at[rows])
        pltpu.sync_copy(us1v, us1_s.at[rows])

        @pl.when(c == 0)
        def _():
            pltpu.sync_copy(ndv, nd_out.at[rows])
        plsc.subcore_barrier()

        wid = s * 2 + c
        B3 = EP // 32
        pltpu.sync_copy(ei_hbm.at[pl.ds(wid * B3, B3)], sidx3)
        pltpu.sync_copy(ei_hbm.at[pl.ds(EP + wid * B3, B3)], didx3)
        pltpu.sync_copy(us0_s.at[sidx3], msg0)
        pltpu.sync_copy(us1_s.at[sidx3], msg1)
        pltpu.sync_copy(msg0, agg0_s.at[didx3], add=True)
        pltpu.sync_copy(msg1, agg1_s.at[didx3], add=True)
        plsc.subcore_barrier()

        pltpu.sync_copy(agg0_s.at[rows], agg_out.at[c, 0, rows])
        pltpu.sync_copy(agg1_s.at[rows], agg_out.at[c, 1, rows])

    return sc_kernel


def kernel(x, edge_index, W_est, b_est, W_est_fc, b_est_fc, W_gnn, b_gnn,
           W_cls, b_cls):
    N, D = x.shape
    E = edge_index.shape[1]

    NP = 256 * ((N + 255) // 256)
    EP = 256 * ((E + 255) // 256)
    if EP > E and NP == N:
        NP += 256

    npad = EP - E
    if npad:
        pad = N + (jnp.arange(npad, dtype=jnp.int32) % (NP - N))
        ei = jnp.concatenate([edge_index, jnp.stack([pad, pad])], axis=1)
    else:
        ei = edge_index

    BR = 2048
    u = pl.pallas_call(
        _u_body,
        grid=(NP // BR,),
        in_specs=[
            pl.BlockSpec((BR, D), lambda i: (i, 0)),
            pl.BlockSpec((D, D), lambda i: (0, 0)),
            pl.BlockSpec((1, D), lambda i: (0, 0)),
            pl.BlockSpec((D, D), lambda i: (0, 0)),
            pl.BlockSpec((1, D), lambda i: (0, 0)),
        ],
        out_specs=pl.BlockSpec((2, BR), lambda i: (0, i)),
        out_shape=jax.ShapeDtypeStruct((2, NP), jnp.float32),
    )(x, W_est, W_est_fc.reshape(1, -1), W_gnn, W_cls.reshape(1, -1))

    ones = jnp.ones((EP // 16,), jnp.float32)
    zeros = jnp.zeros((NP,), jnp.float32)
    agg, nd = _make_sc_kernel(NP, EP)(u, ei.reshape(-1), ones, zeros)

    out2 = pl.pallas_call(
        _fin_body,
        out_shape=jax.ShapeDtypeStruct((2, NP), jnp.float32),
    )(agg, nd, b_est.reshape(1, -1), W_est_fc.reshape(1, -1),
      b_est_fc.reshape(1, 1), b_gnn.reshape(1, -1), W_cls.reshape(1, -1),
      b_cls.reshape(1, 1))

    return (out2[1, :N, None], out2[0, :N, None])


# speedup vs baseline: 1.5777x; 1.1800x over previous
import functools

import jax
import jax.numpy as jnp
from jax import lax
from jax.experimental import pallas as pl
from jax.experimental.pallas import tpu as pltpu
from jax.experimental.pallas import tpu_sc as plsc

_HIGH = jax.lax.Precision.HIGHEST


def _u_body(x_ref, we_ref, wef_ref, wg_ref, wc_ref, u_ref):
    w_s = jax.lax.dot_general(
        wef_ref[...], we_ref[...], (((1,), (1,)), ((), ())), precision=_HIGH)
    w_y = jax.lax.dot_general(
        wc_ref[...], wg_ref[...], (((1,), (1,)), ((), ())), precision=_HIGH)
    w2t = jnp.concatenate([w_s, w_y], axis=0)
    u_ref[...] = jax.lax.dot_general(
        w2t, x_ref[...], (((1,), (1,)), ((), ())), precision=_HIGH)


def _norm_body(deg_ref, u_ref, us0_ref, us1_ref, nd_ref):
    dego = deg_ref[0, 0] + deg_ref[1, 0]
    degi = deg_ref[0, 1] + deg_ref[1, 1]
    ns = jnp.where(dego > 0.5, jax.lax.rsqrt(dego), 0.0)
    nd_ref[...] = jnp.where(degi > 0.5, jax.lax.rsqrt(degi), 0.0)
    us0_ref[...] = u_ref[0] * ns
    us1_ref[...] = u_ref[1] * ns


def _fin_body(agg_ref, nd_ref, be_ref, wef_ref, bef_ref, bg_ref, wc_ref,
              bc_ref, out_ref):
    bias_s = jnp.sum(be_ref[0, :] * wef_ref[0, :]) + bef_ref[0, 0]
    bias_y = jnp.sum(bg_ref[0, :] * wc_ref[0, :]) + bc_ref[0, 0]
    nd = nd_ref[...]
    o_s = (agg_ref[0, 0] + agg_ref[1, 0]) * nd + bias_s
    o_y = (agg_ref[0, 1] + agg_ref[1, 1]) * nd + bias_y
    out_ref[...] = jnp.stack([o_s, o_y])


def _make_deg_kernel(NP, EP):
    R = NP // 16
    B = EP // 32
    mesh = plsc.VectorSubcoreMesh(core_axis_name="c", subcore_axis_name="s")

    @functools.partial(
        pl.kernel,
        out_type=jax.ShapeDtypeStruct((2, 2, NP), jnp.float32),
        mesh=mesh,
        scratch_types=[
            pltpu.VMEM((B,), jnp.int32),
            pltpu.VMEM((B,), jnp.int32),
            pltpu.VMEM((B,), jnp.float32),
            pltpu.VMEM_SHARED((NP,), jnp.float32),
            pltpu.VMEM_SHARED((NP,), jnp.float32),
        ],
    )
    def deg_kernel(ei_hbm, ones_hbm, zeros_hbm, deg_out,
                   sidx, didx, onesv, dego_s, degi_s):
        c = lax.axis_index("c")
        s = lax.axis_index("s")
        rows = pl.ds(s * R, R)
        wid = s * 2 + c
        pltpu.sync_copy(zeros_hbm.at[rows], dego_s.at[rows])
        pltpu.sync_copy(zeros_hbm.at[rows], degi_s.at[rows])
        pltpu.sync_copy(ei_hbm.at[pl.ds(wid * B, B)], sidx)
        pltpu.sync_copy(ei_hbm.at[pl.ds(EP + wid * B, B)], didx)
        pltpu.sync_copy(ones_hbm, onesv)
        plsc.subcore_barrier()
        pltpu.sync_copy(onesv, dego_s.at[sidx], add=True)
        pltpu.sync_copy(onesv, degi_s.at[didx], add=True)
        plsc.subcore_barrier()
        pltpu.sync_copy(dego_s.at[rows], deg_out.at[c, 0, rows])
        pltpu.sync_copy(degi_s.at[rows], deg_out.at[c, 1, rows])

    return deg_kernel


def _make_msg_kernel(NP, EP):
    R = NP // 16
    B = EP // 32
    mesh = plsc.VectorSubcoreMesh(core_axis_name="c", subcore_axis_name="s")

    @functools.partial(
        pl.kernel,
        out_type=jax.ShapeDtypeStruct((2, 2, NP), jnp.float32),
        mesh=mesh,
        scratch_types=[
            pltpu.VMEM((B,), jnp.int32),
            pltpu.VMEM((B,), jnp.int32),
            pltpu.VMEM((B,), jnp.float32),
            pltpu.VMEM((B,), jnp.float32),
            pltpu.VMEM_SHARED((NP,), jnp.float32),
            pltpu.VMEM_SHARED((NP,), jnp.float32),
            pltpu.VMEM_SHARED((NP,), jnp.float32),
            pltpu.VMEM_SHARED((NP,), jnp.float32),
        ],
    )
    def msg_kernel(us0_hbm, us1_hbm, ei_hbm, zeros_hbm, agg_out,
                   sidx, didx, msg0, msg1, us0_s, us1_s, agg0_s, agg1_s):
        c = lax.axis_index("c")
        s = lax.axis_index("s")
        rows = pl.ds(s * R, R)
        wid = s * 2 + c
        pltpu.sync_copy(us0_hbm.at[rows], us0_s.at[rows])
        pltpu.sync_copy(us1_hbm.at[rows], us1_s.at[rows])
        pltpu.sync_copy(zeros_hbm.at[rows], agg0_s.at[rows])
        pltpu.sync_copy(zeros_hbm.at[rows], agg1_s.at[rows])
        pltpu.sync_copy(ei_hbm.at[pl.ds(wid * B, B)], sidx)
        pltpu.sync_copy(ei_hbm.at[pl.ds(EP + wid * B, B)], didx)
        plsc.subcore_barrier()
        pltpu.sync_copy(us0_s.at[sidx], msg0)
        pltpu.sync_copy(us1_s.at[sidx], msg1)
        pltpu.sync_copy(msg0, agg0_s.at[didx], add=True)
        pltpu.sync_copy(msg1, agg1_s.at[didx], add=True)
        plsc.subcore_barrier()
        pltpu.sync_copy(agg0_s.at[rows], agg_out.at[c, 0, rows])
        pltpu.sync_copy(agg1_s.at[rows], agg_out.at[c, 1, rows])

    return msg_kernel


def kernel(x, edge_index, W_est, b_est, W_est_fc, b_est_fc, W_gnn, b_gnn,
           W_cls, b_cls):
    N, D = x.shape
    E = edge_index.shape[1]

    NP = 256 * ((N + 255) // 256)
    EP = 256 * ((E + 255) // 256)
    if EP > E and NP == N:
        NP += 256

    npad = EP - E
    if npad:
        pad = N + (jnp.arange(npad, dtype=jnp.int32) % (NP - N))
        ei = jnp.concatenate([edge_index, jnp.stack([pad, pad])], axis=1)
    else:
        ei = edge_index
    eif = ei.reshape(-1)

    BR = 2048
    u = pl.pallas_call(
        _u_body,
        grid=(NP // BR,),
        in_specs=[
            pl.BlockSpec((BR, D), lambda i: (i, 0)),
            pl.BlockSpec((D, D), lambda i: (0, 0)),
            pl.BlockSpec((1, D), lambda i: (0, 0)),
            pl.BlockSpec((D, D), lambda i: (0, 0)),
            pl.BlockSpec((1, D), lambda i: (0, 0)),
        ],
        out_specs=pl.BlockSpec((2, BR), lambda i: (0, i)),
        out_shape=jax.ShapeDtypeStruct((2, NP), jnp.float32),
    )(x, W_est, W_est_fc.reshape(1, -1), W_gnn, W_cls.reshape(1, -1))

    ones = jnp.ones((EP // 32,), jnp.float32)
    zeros = jnp.zeros((NP,), jnp.float32)

    deg = _make_deg_kernel(NP, EP)(eif, ones, zeros)

    us0, us1, nd = pl.pallas_call(
        _norm_body,
        out_shape=[jax.ShapeDtypeStruct((NP,), jnp.float32)] * 3,
    )(deg, u)

    agg = _make_msg_kernel(NP, EP)(us0, us1, eif, zeros)

    out2 = pl.pallas_call(
        _fin_body,
        out_shape=jax.ShapeDtypeStruct((2, NP), jnp.float32),
    )(agg, nd, b_est.reshape(1, -1), W_est_fc.reshape(1, -1),
      b_est_fc.reshape(1, 1), b_gnn.reshape(1, -1), W_cls.reshape(1, -1),
      b_cls.reshape(1, 1))

    return (out2[1, :N, None], out2[0, :N, None])


# speedup vs baseline: 1.7821x; 1.1296x over previous
import functools

import jax
import jax.numpy as jnp
from jax import lax
from jax.experimental import pallas as pl
from jax.experimental.pallas import tpu as pltpu
from jax.experimental.pallas import tpu_sc as plsc

_HIGH = jax.lax.Precision.HIGHEST


def _u_body(x_ref, we_ref, wef_ref, wg_ref, wc_ref, u_ref):
    w_s = jax.lax.dot_general(
        wef_ref[...], we_ref[...], (((1,), (1,)), ((), ())), precision=_HIGH)
    w_y = jax.lax.dot_general(
        wc_ref[...], wg_ref[...], (((1,), (1,)), ((), ())), precision=_HIGH)
    w2t = jnp.concatenate([w_s, w_y], axis=0)
    u_ref[...] = jax.lax.dot_general(
        w2t, x_ref[...], (((1,), (1,)), ((), ())), precision=_HIGH)


def _norm_body(deg_ref, u_ref, us0_ref, us1_ref, nd_ref):
    dego = deg_ref[0, 0] + deg_ref[1, 0]
    degi = deg_ref[0, 1] + deg_ref[1, 1]
    ns = jnp.where(dego > 0.5, jax.lax.rsqrt(dego), 0.0)
    nd_ref[...] = jnp.where(degi > 0.5, jax.lax.rsqrt(degi), 0.0)
    us0_ref[...] = u_ref[0] * ns
    us1_ref[...] = u_ref[1] * ns


def _fin_body(agg_ref, nd_ref, be_ref, wef_ref, bef_ref, bg_ref, wc_ref,
              bc_ref, out_ref):
    bias_s = jnp.sum(be_ref[0, :] * wef_ref[0, :]) + bef_ref[0, 0]
    bias_y = jnp.sum(bg_ref[0, :] * wc_ref[0, :]) + bc_ref[0, 0]
    nd = nd_ref[...]
    o_s = (agg_ref[0, 0] + agg_ref[1, 0]) * nd + bias_s
    o_y = (agg_ref[0, 1] + agg_ref[1, 1]) * nd + bias_y
    out_ref[...] = jnp.stack([o_s, o_y])


def _make_deg_kernel(NP, EP):
    R = NP // 16
    B = EP // 32
    mesh = plsc.VectorSubcoreMesh(core_axis_name="c", subcore_axis_name="s")

    @functools.partial(
        pl.kernel,
        out_type=jax.ShapeDtypeStruct((2, 2, NP), jnp.float32),
        mesh=mesh,
        scratch_types=[
            pltpu.VMEM((B,), jnp.int32),
            pltpu.VMEM((B,), jnp.int32),
            pltpu.VMEM((B,), jnp.float32),
            pltpu.VMEM_SHARED((NP,), jnp.float32),
            pltpu.VMEM_SHARED((NP,), jnp.float32),
            pltpu.SemaphoreType.DMA,
            pltpu.SemaphoreType.DMA,
        ],
    )
    def deg_kernel(ei_hbm, ones_hbm, zeros_hbm, deg_out,
                   sidx, didx, onesv, dego_s, degi_s, sem0, sem1):
        c = lax.axis_index("c")
        s = lax.axis_index("s")
        rows = pl.ds(s * R, R)
        wid = s * 2 + c
        st = [
            pltpu.async_copy(zeros_hbm.at[rows], dego_s.at[rows], sem0),
            pltpu.async_copy(zeros_hbm.at[rows], degi_s.at[rows], sem0),
            pltpu.async_copy(ei_hbm.at[pl.ds(wid * B, B)], sidx, sem0),
            pltpu.async_copy(ei_hbm.at[pl.ds(EP + wid * B, B)], didx, sem0),
            pltpu.async_copy(ones_hbm, onesv, sem0),
        ]
        for cp in st:
            cp.wait()
        plsc.subcore_barrier()
        a0 = pltpu.async_copy(onesv, dego_s.at[sidx], sem0, add=True)
        a1 = pltpu.async_copy(onesv, degi_s.at[didx], sem1, add=True)
        a0.wait()
        a1.wait()
        plsc.subcore_barrier()
        o0 = pltpu.async_copy(dego_s.at[rows], deg_out.at[c, 0, rows], sem0)
        o1 = pltpu.async_copy(degi_s.at[rows], deg_out.at[c, 1, rows], sem1)
        o0.wait()
        o1.wait()

    return deg_kernel


def _make_msg_kernel(NP, EP):
    R = NP // 16
    B = EP // 32
    mesh = plsc.VectorSubcoreMesh(core_axis_name="c", subcore_axis_name="s")

    @functools.partial(
        pl.kernel,
        out_type=jax.ShapeDtypeStruct((2, 2, NP), jnp.float32),
        mesh=mesh,
        scratch_types=[
            pltpu.VMEM((B,), jnp.int32),
            pltpu.VMEM((B,), jnp.int32),
            pltpu.VMEM((B,), jnp.float32),
            pltpu.VMEM((B,), jnp.float32),
            pltpu.VMEM_SHARED((NP,), jnp.float32),
            pltpu.VMEM_SHARED((NP,), jnp.float32),
            pltpu.VMEM_SHARED((NP,), jnp.float32),
            pltpu.VMEM_SHARED((NP,), jnp.float32),
            pltpu.SemaphoreType.DMA,
            pltpu.SemaphoreType.DMA,
            pltpu.SemaphoreType.DMA,
            pltpu.SemaphoreType.DMA,
        ],
    )
    def msg_kernel(us0_hbm, us1_hbm, ei_hbm, zeros_hbm, agg_out,
                   sidx, didx, msg0, msg1, us0_s, us1_s, agg0_s, agg1_s,
                   sem0, sem1, sem2, sem3):
        c = lax.axis_index("c")
        s = lax.axis_index("s")
        rows = pl.ds(s * R, R)
        wid = s * 2 + c
        st = [
            pltpu.async_copy(us0_hbm.at[rows], us0_s.at[rows], sem0),
            pltpu.async_copy(us1_hbm.at[rows], us1_s.at[rows], sem0),
            pltpu.async_copy(zeros_hbm.at[rows], agg0_s.at[rows], sem0),
            pltpu.async_copy(zeros_hbm.at[rows], agg1_s.at[rows], sem0),
            pltpu.async_copy(ei_hbm.at[pl.ds(wid * B, B)], sidx, sem0),
            pltpu.async_copy(ei_hbm.at[pl.ds(EP + wid * B, B)], didx, sem0),
        ]
        for cp in st:
            cp.wait()
        plsc.subcore_barrier()
        g0 = pltpu.async_copy(us0_s.at[sidx], msg0, sem0)
        g1 = pltpu.async_copy(us1_s.at[sidx], msg1, sem1)
        g0.wait()
        a0 = pltpu.async_copy(msg0, agg0_s.at[didx], sem2, add=True)
        g1.wait()
        a1 = pltpu.async_copy(msg1, agg1_s.at[didx], sem3, add=True)
        a0.wait()
        a1.wait()
        plsc.subcore_barrier()
        o0 = pltpu.async_copy(agg0_s.at[rows], agg_out.at[c, 0, rows], sem0)
        o1 = pltpu.async_copy(agg1_s.at[rows], agg_out.at[c, 1, rows], sem1)
        o0.wait()
        o1.wait()

    return msg_kernel


def kernel(x, edge_index, W_est, b_est, W_est_fc, b_est_fc, W_gnn, b_gnn,
           W_cls, b_cls):
    N, D = x.shape
    E = edge_index.shape[1]

    NP = 256 * ((N + 255) // 256)
    EP = 256 * ((E + 255) // 256)
    if EP > E and NP == N:
        NP += 256

    npad = EP - E
    if npad:
        pad = N + (jnp.arange(npad, dtype=jnp.int32) % (NP - N))
        ei = jnp.concatenate([edge_index, jnp.stack([pad, pad])], axis=1)
    else:
        ei = edge_index
    eif = ei.reshape(-1)

    BR = 2048
    u = pl.pallas_call(
        _u_body,
        grid=(NP // BR,),
        in_specs=[
            pl.BlockSpec((BR, D), lambda i: (i, 0)),
            pl.BlockSpec((D, D), lambda i: (0, 0)),
            pl.BlockSpec((1, D), lambda i: (0, 0)),
            pl.BlockSpec((D, D), lambda i: (0, 0)),
            pl.BlockSpec((1, D), lambda i: (0, 0)),
        ],
        out_specs=pl.BlockSpec((2, BR), lambda i: (0, i)),
        out_shape=jax.ShapeDtypeStruct((2, NP), jnp.float32),
    )(x, W_est, W_est_fc.reshape(1, -1), W_gnn, W_cls.reshape(1, -1))

    ones = jnp.ones((EP // 32,), jnp.float32)
    zeros = jnp.zeros((NP,), jnp.float32)

    deg = _make_deg_kernel(NP, EP)(eif, ones, zeros)

    us0, us1, nd = pl.pallas_call(
        _norm_body,
        out_shape=[jax.ShapeDtypeStruct((NP,), jnp.float32)] * 3,
    )(deg, u)

    agg = _make_msg_kernel(NP, EP)(us0, us1, eif, zeros)

    out2 = pl.pallas_call(
        _fin_body,
        out_shape=jax.ShapeDtypeStruct((2, NP), jnp.float32),
    )(agg, nd, b_est.reshape(1, -1), W_est_fc.reshape(1, -1),
      b_est_fc.reshape(1, 1), b_gnn.reshape(1, -1), W_cls.reshape(1, -1),
      b_cls.reshape(1, 1))

    return (out2[1, :N, None], out2[0, :N, None])


# speedup vs baseline: 1.8545x; 1.0406x over previous
import functools

import jax
import jax.numpy as jnp
from jax import lax
from jax.experimental import pallas as pl
from jax.experimental.pallas import tpu as pltpu
from jax.experimental.pallas import tpu_sc as plsc

_HIGH = jax.lax.Precision.HIGHEST


def _u_body(x_ref, we_ref, wef_ref, wg_ref, wc_ref, u_ref):
    w_s = jax.lax.dot_general(
        wef_ref[...], we_ref[...], (((1,), (1,)), ((), ())), precision=_HIGH)
    w_y = jax.lax.dot_general(
        wc_ref[...], wg_ref[...], (((1,), (1,)), ((), ())), precision=_HIGH)
    w2t = jnp.concatenate([w_s, w_y], axis=0)
    u_ref[...] = jax.lax.dot_general(
        w2t, x_ref[...], (((1,), (1,)), ((), ())), precision=_HIGH)


def _fin_body(agg_ref, nd_ref, be_ref, wef_ref, bef_ref, bg_ref, wc_ref,
              bc_ref, out_ref):
    bias_s = jnp.sum(be_ref[0, :] * wef_ref[0, :]) + bef_ref[0, 0]
    bias_y = jnp.sum(bg_ref[0, :] * wc_ref[0, :]) + bc_ref[0, 0]
    nd = nd_ref[...]
    o_s = (agg_ref[0, 0] + agg_ref[1, 0]) * nd + bias_s
    o_y = (agg_ref[0, 1] + agg_ref[1, 1]) * nd + bias_y
    out_ref[...] = jnp.stack([o_s, o_y])


def _rsqrt16(x):
    i = lax.bitcast_convert_type(x, jnp.int32)
    y = lax.bitcast_convert_type(jnp.int32(0x5F3759DF) - (i >> 1), jnp.float32)
    for _ in range(3):
        y = y * (1.5 - 0.5 * x * y * y)
    return jnp.where(x > 0.5, y, 0.0)


def _make_deg_kernel(NP, EP):
    R = NP // 16
    B = EP // 32
    mesh = plsc.VectorSubcoreMesh(core_axis_name="c", subcore_axis_name="s")

    @functools.partial(
        pl.kernel,
        out_type=jax.ShapeDtypeStruct((2, 2, NP), jnp.float32),
        mesh=mesh,
        scratch_types=[
            pltpu.VMEM((B,), jnp.int32),
            pltpu.VMEM((B,), jnp.int32),
            pltpu.VMEM((B,), jnp.float32),
            pltpu.VMEM((R,), jnp.float32),
            pltpu.VMEM_SHARED((NP,), jnp.float32),
            pltpu.VMEM_SHARED((NP,), jnp.float32),
            pltpu.SemaphoreType.DMA,
            pltpu.SemaphoreType.DMA,
        ],
    )
    def deg_kernel(ei_hbm, deg_out,
                   sidx, didx, onesv, zv, dego_s, degi_s, sem0, sem1):
        c = lax.axis_index("c")
        s = lax.axis_index("s")
        rows = pl.ds(s * R, R)
        wid = s * 2 + c
        i0 = pltpu.async_copy(ei_hbm.at[pl.ds(wid * B, B)], sidx, sem0)
        i1 = pltpu.async_copy(ei_hbm.at[pl.ds(EP + wid * B, B)], didx, sem0)
        one16 = jnp.ones((16,), jnp.float32)
        zero16 = jnp.zeros((16,), jnp.float32)

        def fill(g, carry):
            onesv[pl.ds(g * 16, 16)] = one16
            return carry
        lax.fori_loop(0, B // 16, fill, 0)

        def fillz(g, carry):
            zv[pl.ds(g * 16, 16)] = zero16
            return carry
        lax.fori_loop(0, R // 16, fillz, 0)
        z0 = pltpu.async_copy(zv, dego_s.at[rows], sem0)
        z1 = pltpu.async_copy(zv, degi_s.at[rows], sem0)
        for cp in (i0, i1, z0, z1):
            cp.wait()
        plsc.subcore_barrier()
        a0 = pltpu.async_copy(onesv, dego_s.at[sidx], sem0, add=True)
        a1 = pltpu.async_copy(onesv, degi_s.at[didx], sem1, add=True)
        a0.wait()
        a1.wait()
        plsc.subcore_barrier()
        o0 = pltpu.async_copy(dego_s.at[rows], deg_out.at[c, 0, rows], sem0)
        o1 = pltpu.async_copy(degi_s.at[rows], deg_out.at[c, 1, rows], sem1)
        o0.wait()
        o1.wait()

    return deg_kernel


def _make_msg_kernel(NP, EP):
    R = NP // 16
    B = EP // 32
    mesh = plsc.VectorSubcoreMesh(core_axis_name="c", subcore_axis_name="s")

    @functools.partial(
        pl.kernel,
        out_type=[
            jax.ShapeDtypeStruct((2, 2, NP), jnp.float32),
            jax.ShapeDtypeStruct((NP,), jnp.float32),
        ],
        mesh=mesh,
        scratch_types=[
            pltpu.VMEM((B,), jnp.int32),
            pltpu.VMEM((B,), jnp.int32),
            pltpu.VMEM((B,), jnp.float32),
            pltpu.VMEM((B,), jnp.float32),
            pltpu.VMEM((R,), jnp.float32),
            pltpu.VMEM((R,), jnp.float32),
            pltpu.VMEM((R,), jnp.float32),
            pltpu.VMEM((R,), jnp.float32),
            pltpu.VMEM((R,), jnp.float32),
            pltpu.VMEM((R,), jnp.float32),
            pltpu.VMEM((R,), jnp.float32),
            pltpu.VMEM((R,), jnp.float32),
            pltpu.VMEM((R,), jnp.float32),
            pltpu.VMEM_SHARED((NP,), jnp.float32),
            pltpu.VMEM_SHARED((NP,), jnp.float32),
            pltpu.VMEM_SHARED((NP,), jnp.float32),
            pltpu.VMEM_SHARED((NP,), jnp.float32),
            pltpu.SemaphoreType.DMA,
            pltpu.SemaphoreType.DMA,
            pltpu.SemaphoreType.DMA,
            pltpu.SemaphoreType.DMA,
        ],
    )
    def msg_kernel(u_hbm, deg_hbm, ei_hbm, agg_out, nd_out,
                   sidx, didx, msg0, msg1, da0, da1, db0, db1,
                   u0v, u1v, us0v, us1v, ndv,
                   us0_s, us1_s, agg0_s, agg1_s, sem0, sem1, sem2, sem3):
        c = lax.axis_index("c")
        s = lax.axis_index("s")
        rows = pl.ds(s * R, R)
        wid = s * 2 + c
        st = [
            pltpu.async_copy(ei_hbm.at[pl.ds(wid * B, B)], sidx, sem0),
            pltpu.async_copy(ei_hbm.at[pl.ds(EP + wid * B, B)], didx, sem0),
            pltpu.async_copy(deg_hbm.at[0, 0, rows], da0, sem1),
            pltpu.async_copy(deg_hbm.at[1, 0, rows], da1, sem1),
            pltpu.async_copy(deg_hbm.at[0, 1, rows], db0, sem1),
            pltpu.async_copy(deg_hbm.at[1, 1, rows], db1, sem1),
            pltpu.async_copy(u_hbm.at[0, rows], u0v, sem1),
            pltpu.async_copy(u_hbm.at[1, rows], u1v, sem1),
        ]
        for cp in st[2:]:
            cp.wait()

        zero16 = jnp.zeros((16,), jnp.float32)

        def norm_g(g, carry):
            sl = pl.ds(g * 16, 16)
            ns = _rsqrt16(da0[sl] + da1[sl])
            ndv[sl] = _rsqrt16(db0[sl] + db1[sl])
            us0v[sl] = u0v[sl] * ns
            us1v[sl] = u1v[sl] * ns
            return carry
        lax.fori_loop(0, R // 16, norm_g, 0)

        w0 = pltpu.async_copy(us0v, us0_s.at[rows], sem1)
        w1 = pltpu.async_copy(us1v, us1_s.at[rows], sem1)

        @pl.when(c == 0)
        def _():
            pltpu.sync_copy(ndv, nd_out.at[rows])

        def fillz(g, carry):
            da0[pl.ds(g * 16, 16)] = zero16
            return carry
        lax.fori_loop(0, R // 16, fillz, 0)
        z0 = pltpu.async_copy(da0, agg0_s.at[rows], sem1)
        z1 = pltpu.async_copy(da0, agg1_s.at[rows], sem1)
        for cp in (st[0], st[1], w0, w1, z0, z1):
            cp.wait()
        plsc.subcore_barrier()
        g0 = pltpu.async_copy(us0_s.at[sidx], msg0, sem0)
        g1 = pltpu.async_copy(us1_s.at[sidx], msg1, sem1)
        g0.wait()
        a0 = pltpu.async_copy(msg0, agg0_s.at[didx], sem2, add=True)
        g1.wait()
        a1 = pltpu.async_copy(msg1, agg1_s.at[didx], sem3, add=True)
        a0.wait()
        a1.wait()
        plsc.subcore_barrier()
        o0 = pltpu.async_copy(agg0_s.at[rows], agg_out.at[c, 0, rows], sem0)
        o1 = pltpu.async_copy(agg1_s.at[rows], agg_out.at[c, 1, rows], sem1)
        o0.wait()
        o1.wait()

    return msg_kernel


def kernel(x, edge_index, W_est, b_est, W_est_fc, b_est_fc, W_gnn, b_gnn,
           W_cls, b_cls):
    N, D = x.shape
    E = edge_index.shape[1]

    NP = 256 * ((N + 255) // 256)
    EP = 256 * ((E + 255) // 256)
    if EP > E and NP == N:
        NP += 256

    npad = EP - E
    if npad:
        pad = N + (jnp.arange(npad, dtype=jnp.int32) % (NP - N))
        ei = jnp.concatenate([edge_index, jnp.stack([pad, pad])], axis=1)
    else:
        ei = edge_index
    eif = ei.reshape(-1)

    BR = 2048
    u = pl.pallas_call(
        _u_body,
        grid=(NP // BR,),
        in_specs=[
            pl.BlockSpec((BR, D), lambda i: (i, 0)),
            pl.BlockSpec((D, D), lambda i: (0, 0)),
            pl.BlockSpec((1, D), lambda i: (0, 0)),
            pl.BlockSpec((D, D), lambda i: (0, 0)),
            pl.BlockSpec((1, D), lambda i: (0, 0)),
        ],
        out_specs=pl.BlockSpec((2, BR), lambda i: (0, i)),
        out_shape=jax.ShapeDtypeStruct((2, NP), jnp.float32),
    )(x, W_est, W_est_fc.reshape(1, -1), W_gnn, W_cls.reshape(1, -1))

    deg = _make_deg_kernel(NP, EP)(eif)

    agg, nd = _make_msg_kernel(NP, EP)(u, deg, eif)

    out2 = pl.pallas_call(
        _fin_body,
        out_shape=jax.ShapeDtypeStruct((2, NP), jnp.float32),
    )(agg, nd, b_est.reshape(1, -1), W_est_fc.reshape(1, -1),
      b_est_fc.reshape(1, 1), b_gnn.reshape(1, -1), W_cls.reshape(1, -1),
      b_cls.reshape(1, 1))

    return (out2[1, :N, None], out2[0, :N, None])
